# baseline (speedup 1.0000x reference)
import functools

import jax
import jax.numpy as jnp
from jax import lax
from jax.experimental import pallas as pl
from jax.experimental.pallas import tpu as pltpu
from jax.experimental.pallas import tpu_sc as plsc

_NUM_CORES = 2
_NUM_SUBCORES = 16
_LANES = 16
_NPHASE = 16
_GROUPS = 250


def _build_sc_kernel(q_len, v_len, vocab, dim):
  mp = (vocab - 1) // 2
  assert _NPHASE == _LANES
  t16_vlo = -(_LANES - 1)
  t16_rows = (vocab - 1 - t16_vlo + 1) * dim
  toff = -(q_len - 1 - mp)
  cols = _GROUPS * _LANES
  full_cols = v_len + (q_len - _NPHASE)
  trim = full_cols - cols
  assert 0 <= trim < v_len and trim % _NPHASE == 0
  ph_per_sc = _NPHASE // _NUM_CORES
  d_half = dim // 2
  halves = 4
  d_q = d_half // halves
  gpad = 256
  rows_pad = d_q * gpad
  n_workers = _NUM_CORES * _NUM_SUBCORES
  assert q_len % n_workers == 0
  rows_per_worker = q_len // n_workers
  mblocks_per_tile = rows_per_worker // (_NPHASE // _NUM_CORES)

  mesh = plsc.VectorSubcoreMesh(
      core_axis_name="c", subcore_axis_name="s")

  @functools.partial(
      pl.kernel,
      out_type=[
          jax.ShapeDtypeStruct((q_len, dim, v_len), jnp.float32),
          jax.ShapeDtypeStruct((_NPHASE, dim, cols), jnp.float32),
      ],
      mesh=mesh,
      scratch_types=[
          pltpu.VMEM((rows_pad // 128, 128), jnp.int32),
          pltpu.VMEM((rows_pad, _LANES), jnp.float32),
          pltpu.VMEM((d_q * cols,), jnp.float32),
          pltpu.VMEM_SHARED((ph_per_sc, dim, cols), jnp.float32),
          pltpu.SemaphoreType.DMA,
      ],
      compiler_params=pltpu.CompilerParams(use_tc_tiling_on_sc=False),
  )
  def body(t16_hbm, out_hbm, stage_hbm, idx_v, strips_v, flat_v, cmast, sem):
    c = lax.axis_index("c")
    s = lax.axis_index("s")

    pl_loc = s % ph_per_sc
    d0_tile = (s // ph_per_sc) * d_half
    p16 = c * ph_per_sc + pl_loc
    lanes = lax.iota(jnp.int32, _LANES)
    for h in range(halves):
      d0 = d0_tile + h * d_q
      for r16 in range(rows_pad // _LANES):
        strip = lanes + r16 * _LANES
        ddv = lax.shift_right_logical(strip, 8)
        gv = lax.bitwise_and(strip, gpad - 1)
        v = jnp.clip(gv * _LANES + p16 + toff, t16_vlo, 2 * mp)
        row = (v - t16_vlo) * dim + d0 + ddv
        row = jnp.where(gv < _GROUPS, row, 0)
        idx_v[r16 // 8, pl.ds((r16 % 8) * _LANES, _LANES)] = row
      gathers = [
          pltpu.async_copy(
              t16_hbm.at[idx_v.at[ch]],
              strips_v.at[pl.ds(ch * 128, 128)],
              sem)
          for ch in range(rows_pad // 128)
      ]
      for cp in gathers:
        cp.wait()
      for dd in range(d_q):
        def flat_step(g, _, dd=dd):
          flat_v[pl.ds(dd * cols + g * _LANES, _LANES)] = (
              strips_v[dd * gpad + g, :])
          return _
        lax.fori_loop(0, _GROUPS, flat_step, None)
      stages = [
          pltpu.async_copy(
              flat_v.at[pl.ds(dd * cols, cols)],
              stage_hbm.at[p16, d0 + dd],
              sem)
          for dd in range(d_q)
      ]
      for cp in stages:
        cp.wait()

    plsc.subcore_barrier()

    @pl.when(s == 0)
    def _pull():
      pltpu.sync_copy(stage_hbm.at[pl.ds(c * ph_per_sc, ph_per_sc)], cmast)

    plsc.subcore_barrier()

    row_copies = []
    for mm in range(mblocks_per_tile):
      for pp in range(ph_per_sc):
        pl_r = ph_per_sc - 1 - pp
        i = (s * mblocks_per_tile + mm) * _NPHASE + (8 - 8 * c) + pp
        w0 = q_len - 1 - i
        x0 = pl.multiple_of(
            w0 - (pl_r + ph_per_sc * c), _NPHASE)
        if mm * _NPHASE < trim:
          tail = trim - mm * _NPHASE
          l1 = v_len - tail
          c2 = pl.multiple_of(
              jnp.where(s == 0, 3 * v_len // 2, x0 + l1), _NPHASE)
          row_copies.append(
              pltpu.async_copy(
                  cmast.at[pl_r, :, pl.ds(x0, l1)],
                  out_hbm.at[i, :, pl.ds(0, l1)],
                  sem))
          row_copies.append(
              pltpu.async_copy(
                  cmast.at[pl_r, :, pl.ds(c2, tail)],
                  out_hbm.at[i, :, pl.ds(l1, tail)],
                  sem))
        else:
          row_copies.append(
              pltpu.async_copy(
                  cmast.at[pl_r, :, pl.ds(x0, v_len)],
                  out_hbm.at[i],
                  sem))
    for cp in row_copies:
      cp.wait()

  return body


def kernel(query, value, weight):
  q_len = query.shape[1]
  v_len = value.shape[1]
  vocab, dim = weight.shape
  vlo = -(_LANES - 1)
  vv = jnp.clip(
      jnp.arange(vlo, vocab)[:, None] + jnp.arange(_LANES)[None, :],
      0, vocab - 1)
  t16 = jnp.reshape(
      jnp.transpose(weight[vv], (0, 2, 1)),
      ((vocab - vlo) * dim, _LANES))
  sc = _build_sc_kernel(q_len, v_len, vocab, dim)
  out_t, _ = sc(t16)
  return jnp.swapaxes(out_t, 1, 2)


# speedup vs baseline: 1.0585x; 1.0585x over previous
import functools

import jax
import jax.numpy as jnp
from jax import lax
from jax.experimental import pallas as pl
from jax.experimental.pallas import tpu as pltpu
from jax.experimental.pallas import tpu_sc as plsc

_NUM_CORES = 2
_NUM_SUBCORES = 16
_LANES = 16
_WIDE = 128
_CHUNK = 128
_PHASES = 4


def _build_sc_kernel(q_len, v_len, vocab, dim):
  mp = (vocab - 1) // 2
  group = _WIDE // dim
  assert group * dim == _WIDE and _PHASES == group
  t4_rows = vocab + group - 1
  toff = -(q_len - 1 - mp)
  g_flat = (q_len + v_len - 1) * dim
  rows_per_phase = -(-g_flat // _WIDE)
  assert rows_per_phase % (_NUM_SUBCORES // _PHASES * 2) == 0
  n_workers = _NUM_CORES * _NUM_SUBCORES
  assert q_len % n_workers == 0
  rows_per_worker = q_len // n_workers
  out_mid = v_len * dim // _WIDE
  win_rows = out_mid
  quarters = _NUM_SUBCORES // _PHASES
  chunks_per_tile = rows_per_phase // _CHUNK // quarters

  mesh = plsc.VectorSubcoreMesh(
      core_axis_name="c", subcore_axis_name="s")

  @functools.partial(
      pl.kernel,
      out_type=[
          jax.ShapeDtypeStruct((q_len, out_mid, _WIDE), jnp.float32),
          jax.ShapeDtypeStruct((_PHASES * rows_per_phase, _WIDE),
                               jnp.float32),
      ],
      mesh=mesh,
      scratch_types=[
          pltpu.VMEM((chunks_per_tile, _CHUNK), jnp.int32),
          pltpu.VMEM((chunks_per_tile * _CHUNK, _WIDE), jnp.float32),
          pltpu.VMEM_SHARED((_PHASES * rows_per_phase, _WIDE),
                            jnp.float32),
          pltpu.SemaphoreType.DMA,
      ],
      compiler_params=pltpu.CompilerParams(use_tc_tiling_on_sc=False),
  )
  def body(t4_hbm, out_hbm, stage_hbm, idx_v, buf_v, g_all, sem):
    c = lax.axis_index("c")
    s = lax.axis_index("s")

    phase = s // quarters
    quarter = s % quarters
    k0_tile = quarter * (chunks_per_tile * _CHUNK)
    lanes = lax.iota(jnp.int32, _LANES)
    for ci in range(chunks_per_tile):
      for kk in range(_CHUNK // _LANES):
        k = k0_tile + ci * _CHUNK + kk * _LANES
        vals = jnp.clip((lanes + k) * group + phase + toff,
                        -(group - 1), 2 * mp) + (group - 1)
        idx_v[ci, pl.ds(kk * _LANES, _LANES)] = vals
    gathers = [
        pltpu.async_copy(
            t4_hbm.at[idx_v.at[ci]],
            buf_v.at[pl.ds(ci * _CHUNK, _CHUNK)],
            sem)
        for ci in range(chunks_per_tile)
    ]
    for cp in gathers:
      cp.wait()
    stages = [
        pltpu.async_copy(
            buf_v.at[pl.ds(ci * _CHUNK, _CHUNK)],
            stage_hbm.at[pl.ds(phase * rows_per_phase + k0_tile + ci * _CHUNK,
                               _CHUNK)],
            sem)
        for ci in range(chunks_per_tile)
    ]
    for cp in stages:
      cp.wait()

    plsc.subcore_barrier()

    @pl.when(s == 0)
    def _pull():
      pltpu.sync_copy(stage_hbm, g_all)

    plsc.subcore_barrier()

    wid = s * _NUM_CORES + c
    base = wid * rows_per_worker
    row_copies = []
    for r in range(rows_per_worker):
      i = base + r
      p = (q_len - 1 - r) % _PHASES
      k0 = (q_len - 1 - i - p) // _PHASES
      row_copies.append(
          pltpu.async_copy(
              g_all.at[pl.ds(p * rows_per_phase + k0, win_rows)],
              out_hbm.at[i],
              sem))
    for cp in row_copies:
      cp.wait()

  return body


def kernel(query, value, weight):
  q_len = query.shape[1]
  v_len = value.shape[1]
  vocab, dim = weight.shape
  group = _WIDE // dim
  vidx = jnp.clip(
      jnp.arange(-(group - 1), vocab)[:, None] + jnp.arange(group)[None, :],
      0, vocab - 1)
  t4 = jnp.reshape(weight[vidx], (vocab + group - 1, group * dim))
  sc = _build_sc_kernel(q_len, v_len, vocab, dim)
  out, _ = sc(t4)
  return jnp.reshape(out, (q_len, v_len, dim))
